# baseline (speedup 1.0000x reference)
import functools

import jax
import jax.numpy as jnp
from jax import lax
from jax.experimental import pallas as pl
from jax.experimental.pallas import tpu as pltpu
from jax.experimental.pallas import tpu_sc as plsc

NC = 2
NS = 16
L = 16


def _sc_body(B, C, HW, ML, no, NOBJ,
             ft_hbm, lab_hbm, nobj_hbm,
             src_obj, mask_obj, src_bg, mask_bg, mask_other,
             labv, idxbuf, maskv, nobjv, fbuf, sem_g, sem_s):
    cid = lax.axis_index("c")
    sid = lax.axis_index("s")
    wid = sid * NC + cid
    pltpu.sync_copy(nobj_hbm, nobjv)
    lanes = lax.broadcasted_iota(jnp.int32, (L,), 0)
    nchunk = HW // L
    ccol = C // L

    zf32 = jnp.zeros((L,), jnp.float32)
    for cc in range(ccol):
        fbuf[HW, pl.ds(cc * L, L)] = zf32

    for r in range(2):
        t = wid * 2 + r
        b = t // 8
        chm1 = t % 8
        ch = chm1 + 1
        is_bg = chm1 == 7

        cpf = pltpu.async_copy(ft_hbm.at[pl.ds(b * HW, HW)],
                               fbuf.at[pl.ds(0, HW)], sem_g)
        pltpu.sync_copy(lab_hbm.at[pl.ds((b * NOBJ + ch) * HW, HW)], labv)
        nobj_b = jnp.sum(jnp.where(lanes == b, nobjv[...], 0))
        skip = jnp.logical_and(chm1 < 7, ch > nobj_b)
        keep = jnp.broadcast_to(jnp.logical_not(skip), (L,))

        zfill = jnp.full((L,), HW, jnp.int32)
        for k in range(nchunk):
            idxbuf[pl.ds(k * L, L)] = zfill
        off = jnp.int32(0)
        for k in range(nchunk):
            lab16 = labv[pl.ds(k * L, L)]
            m = jnp.logical_and(lab16 == 1, keep)
            mi = m.astype(jnp.int32)
            pos = plsc.cumsum(mi) - mi + off
            pix = lanes + (k * L)
            plsc.store_scatter(idxbuf, [pos], pix, mask=m)
            off = off + jnp.sum(mi)
        count = off

        for k in range(nchunk):
            i16 = lanes + (k * L)
            maskv[pl.ds(k * L, L)] = jnp.where(
                i16 >= count, jnp.float32(1.0), jnp.float32(0.0))

        cpf.wait()

        def _pack_row(i, _):
            si = idxbuf[pl.ds(i, L)][0]
            for cc in range(ccol):
                fbuf[i, pl.ds(cc * L, L)] = fbuf[si, pl.ds(cc * L, L)]
            return 0

        lax.fori_loop(0, HW, _pack_row, 0)

        o = chm1

        @pl.when(jnp.logical_not(is_bg))
        def _():
            pend = []
            base0 = (b * no + o) * ML
            pend.append(pltpu.async_copy(
                fbuf.at[pl.ds(0, ML)], src_obj.at[pl.ds(base0, ML)], sem_s))
            pend.append(pltpu.async_copy(
                maskv, mask_obj.at[pl.ds(base0, ML)], sem_s))
            for d in range(1, no):
                o2 = lax.rem(o + d, no)
                j = o - (o2 < o).astype(jnp.int32)
                base = (b * no + o2) * (ML * no) + j * ML
                pend.append(pltpu.async_copy(
                    maskv, mask_other.at[pl.ds(base, ML)], sem_s))
            for p in pend:
                p.wait()

        @pl.when(is_bg)
        def _():
            pend = []
            pend.append(pltpu.async_copy(
                fbuf.at[pl.ds(0, ML)], src_bg.at[pl.ds(b * ML, ML)], sem_s))
            pend.append(pltpu.async_copy(
                maskv, mask_bg.at[pl.ds(b * ML, ML)], sem_s))
            for o2 in range(no):
                base = (b * no + o2) * (ML * no) + (no - 1) * ML
                pend.append(pltpu.async_copy(
                    maskv, mask_other.at[pl.ds(base, ML)], sem_s))
            for p in pend:
                p.wait()


def _tc_pack_body(no, ML, nobj_ref, tri_ref, lab_ref, ft_ref, out_ref):
    b = pl.program_id(0)
    s = pl.program_id(1)
    sch = s + 1
    keep = jnp.logical_or(s == no, sch <= nobj_ref[b])
    lab_row = lab_ref[0, pl.ds(sch, 1), :]
    m = jnp.logical_and(lab_row == 1, keep)
    mf = jnp.where(m, jnp.float32(1.0), jnp.float32(0.0))
    excl = jnp.dot(mf, tri_ref[...], preferred_element_type=jnp.float32)
    row_i = lax.broadcasted_iota(jnp.int32, (ML, 1), 0).astype(jnp.float32)
    P = jnp.where(jnp.logical_and(excl == row_i, m),
                  jnp.float32(1.0), jnp.float32(0.0))
    out = jax.lax.dot_general(P.astype(jnp.bfloat16),
                              ft_ref[0].astype(jnp.bfloat16),
                              (((1,), (0,)), ((), ())),
                              preferred_element_type=jnp.float32)
    out_ref[0, 0] = out


def _tc_copy_body(packed_ref, out_ref):
    out_ref[0, 0, 0] = packed_ref[0, 0]


def kernel(feats, label, num_objects):
    B, C, H, W = feats.shape
    HW = H * W
    ML = 256
    NOBJ = label.shape[1]
    no = num_objects.shape[0] - 1

    ft = feats.reshape(B, C, HW).transpose(0, 2, 1).reshape(B * HW, C)
    lab_flat = label.reshape(B * NOBJ * HW).astype(jnp.int32)
    nobj16 = jnp.pad(num_objects.astype(jnp.int32), (0, 16 - B))

    mesh = plsc.VectorSubcoreMesh(core_axis_name="c", subcore_axis_name="s",
                                  num_cores=NC, num_subcores=NS)
    out_type = (
        jax.ShapeDtypeStruct((B * no * ML, C), jnp.float32),
        jax.ShapeDtypeStruct((B * no * ML,), jnp.float32),
        jax.ShapeDtypeStruct((B * ML, C), jnp.float32),
        jax.ShapeDtypeStruct((B * ML,), jnp.float32),
        jax.ShapeDtypeStruct((B * no * ML * no,), jnp.float32),
    )
    scratch_types = [
        pltpu.VMEM((HW,), jnp.int32),
        pltpu.VMEM((HW + L,), jnp.int32),
        pltpu.VMEM((HW,), jnp.float32),
        pltpu.VMEM((16,), jnp.int32),
        pltpu.VMEM((HW + 1, C), jnp.float32),
        pltpu.SemaphoreType.DMA,
        pltpu.SemaphoreType.DMA,
    ]
    body = functools.partial(_sc_body, B, C, HW, ML, no, NOBJ)
    outs = pl.kernel(
        body, out_type=out_type, mesh=mesh,
        scratch_types=scratch_types,
        compiler_params=pltpu.CompilerParams(needs_layout_passes=False),
        name="scribble_pool_sc")(ft, lab_flat, nobj16)
    o1, o2, o3, o4, o6 = outs

    ft3 = ft.reshape(B, HW, C)
    lab3 = label.reshape(B, NOBJ, HW).astype(jnp.int32)
    qi = lax.broadcasted_iota(jnp.int32, (ML, ML), 0)
    pi = lax.broadcasted_iota(jnp.int32, (ML, ML), 1)
    tri = (qi < pi).astype(jnp.float32)
    packed_all = pl.pallas_call(
        functools.partial(_tc_pack_body, no, ML),
        grid=(B, no + 1),
        in_specs=[
            pl.BlockSpec(memory_space=pltpu.SMEM),
            pl.BlockSpec((ML, ML), lambda b, s: (0, 0)),
            pl.BlockSpec((1, NOBJ, HW), lambda b, s: (b, 0, 0)),
            pl.BlockSpec((1, HW, C), lambda b, s: (b, 0, 0)),
        ],
        out_specs=pl.BlockSpec((1, 1, ML, C), lambda b, s: (b, s, 0, 0)),
        out_shape=jax.ShapeDtypeStruct((B, no + 1, ML, C), jnp.float32),
    )(nobj16, tri, lab3, ft3)

    def _src_idx(b, j, o2):
        return (b, j + (j >= o2).astype(jnp.int32), 0, 0)

    src_other5 = pl.pallas_call(
        _tc_copy_body,
        grid=(B, no, no),
        in_specs=[pl.BlockSpec((1, 1, ML, C), _src_idx)],
        out_specs=pl.BlockSpec((1, 1, 1, ML, C),
                               lambda b, j, o2: (b, o2, j, 0, 0)),
        out_shape=jax.ShapeDtypeStruct((B, no, no, ML, C), jnp.float32),
    )(packed_all)

    return (o1.reshape(B * no, ML, C),
            o2.reshape(B * no, ML),
            o3.reshape(B, ML, C),
            o4.reshape(B, ML),
            src_other5.reshape(B * no, ML * no, C),
            o6.reshape(B * no, ML * no))


# speedup vs baseline: 2.0661x; 2.0661x over previous
import functools

import jax
import jax.numpy as jnp
from jax import lax
from jax.experimental import pallas as pl
from jax.experimental.pallas import tpu as pltpu
from jax.experimental.pallas import tpu_sc as plsc

NC = 2
NS = 16
L = 16


def _sc_body(B, C, HW, ML, no, NOBJ,
             ft_hbm, lab_hbm, nobj_hbm,
             src_obj, mask_obj, src_bg, mask_bg, src_other, mask_other,
             labv, idxbuf, maskv, nobjv, fbuf, sem_g, sem_s):
    cid = lax.axis_index("c")
    sid = lax.axis_index("s")
    wid = sid * NC + cid
    pltpu.sync_copy(nobj_hbm, nobjv)
    lanes = lax.broadcasted_iota(jnp.int32, (L,), 0)
    nchunk = HW // L
    ccol = C // L

    zf32 = jnp.zeros((L,), jnp.float32)
    for cc in range(ccol):
        fbuf[HW, pl.ds(cc * L, L)] = zf32

    for r in range(2):
        t = wid * 2 + r
        b = t // 8
        chm1 = t % 8
        ch = chm1 + 1
        is_bg = chm1 == 7

        cpf = pltpu.async_copy(ft_hbm.at[pl.ds(b * HW, HW)],
                               fbuf.at[pl.ds(0, HW)], sem_g)
        pltpu.sync_copy(lab_hbm.at[pl.ds((b * NOBJ + ch) * HW, HW)], labv)
        nobj_b = jnp.sum(jnp.where(lanes == b, nobjv[...], 0))
        skip = jnp.logical_and(chm1 < 7, ch > nobj_b)
        keep = jnp.broadcast_to(jnp.logical_not(skip), (L,))

        zfill = jnp.full((L,), HW, jnp.int32)
        for k in range(nchunk):
            idxbuf[pl.ds(k * L, L)] = zfill
        off = jnp.int32(0)
        for k in range(nchunk):
            lab16 = labv[pl.ds(k * L, L)]
            m = jnp.logical_and(lab16 == 1, keep)
            mi = m.astype(jnp.int32)
            pos = plsc.cumsum(mi) - mi + off
            pix = lanes + (k * L)
            plsc.store_scatter(idxbuf, [pos], pix, mask=m)
            off = off + jnp.sum(mi)
        count = off

        for k in range(nchunk):
            i16 = lanes + (k * L)
            maskv[pl.ds(k * L, L)] = jnp.where(
                i16 >= count, jnp.float32(1.0), jnp.float32(0.0))

        cpf.wait()

        def _pack_chunk(k, _):
            i0 = k * L
            idx16 = idxbuf[pl.ds(i0, L)]
            for kk in range(L):
                si = idx16[kk]
                for cc in range(ccol):
                    fbuf[i0 + kk, pl.ds(cc * L, L)] = fbuf[si, pl.ds(cc * L, L)]
            return 0

        lax.fori_loop(0, nchunk, _pack_chunk, 0)

        o = chm1

        @pl.when(jnp.logical_not(is_bg))
        def _():
            pend = []
            base0 = (b * no + o) * ML
            pend.append(pltpu.async_copy(
                fbuf.at[pl.ds(0, ML)], src_obj.at[pl.ds(base0, ML)], sem_s))
            pend.append(pltpu.async_copy(
                maskv, mask_obj.at[pl.ds(base0, ML)], sem_s))
            for d in range(1, no):
                o2 = lax.rem(o + d, no)
                j = o - (o2 < o).astype(jnp.int32)
                base = (b * no + o2) * (ML * no) + j * ML
                pend.append(pltpu.async_copy(
                    fbuf.at[pl.ds(0, ML)], src_other.at[pl.ds(base, ML)], sem_s))
                pend.append(pltpu.async_copy(
                    maskv, mask_other.at[pl.ds(base, ML)], sem_s))
            for p in pend:
                p.wait()

        @pl.when(is_bg)
        def _():
            pend = []
            pend.append(pltpu.async_copy(
                fbuf.at[pl.ds(0, ML)], src_bg.at[pl.ds(b * ML, ML)], sem_s))
            pend.append(pltpu.async_copy(
                maskv, mask_bg.at[pl.ds(b * ML, ML)], sem_s))
            for o2 in range(no):
                base = (b * no + o2) * (ML * no) + (no - 1) * ML
                pend.append(pltpu.async_copy(
                    fbuf.at[pl.ds(0, ML)], src_other.at[pl.ds(base, ML)], sem_s))
                pend.append(pltpu.async_copy(
                    maskv, mask_other.at[pl.ds(base, ML)], sem_s))
            for p in pend:
                p.wait()


def kernel(feats, label, num_objects):
    B, C, H, W = feats.shape
    HW = H * W
    ML = 256
    NOBJ = label.shape[1]
    no = num_objects.shape[0] - 1

    ft = feats.reshape(B, C, HW).transpose(0, 2, 1).reshape(B * HW, C)
    lab_flat = label.reshape(B * NOBJ * HW).astype(jnp.int32)
    nobj16 = jnp.pad(num_objects.astype(jnp.int32), (0, 16 - B))

    mesh = plsc.VectorSubcoreMesh(core_axis_name="c", subcore_axis_name="s",
                                  num_cores=NC, num_subcores=NS)
    out_type = (
        jax.ShapeDtypeStruct((B * no * ML, C), jnp.float32),
        jax.ShapeDtypeStruct((B * no * ML,), jnp.float32),
        jax.ShapeDtypeStruct((B * ML, C), jnp.float32),
        jax.ShapeDtypeStruct((B * ML,), jnp.float32),
        jax.ShapeDtypeStruct((B * no * ML * no, C), jnp.float32),
        jax.ShapeDtypeStruct((B * no * ML * no,), jnp.float32),
    )
    scratch_types = [
        pltpu.VMEM((HW,), jnp.int32),
        pltpu.VMEM((HW + L,), jnp.int32),
        pltpu.VMEM((HW,), jnp.float32),
        pltpu.VMEM((16,), jnp.int32),
        pltpu.VMEM((HW + 1, C), jnp.float32),
        pltpu.SemaphoreType.DMA,
        pltpu.SemaphoreType.DMA,
    ]
    body = functools.partial(_sc_body, B, C, HW, ML, no, NOBJ)
    outs = pl.kernel(
        body, out_type=out_type, mesh=mesh,
        scratch_types=scratch_types,
        compiler_params=pltpu.CompilerParams(needs_layout_passes=False),
        name="scribble_pool_sc")(ft, lab_flat, nobj16)
    o1, o2, o3, o4, o5, o6 = outs
    return (o1.reshape(B * no, ML, C),
            o2.reshape(B * no, ML),
            o3.reshape(B, ML, C),
            o4.reshape(B, ML),
            o5.reshape(B * no, ML * no, C),
            o6.reshape(B * no, ML * no))


# speedup vs baseline: 2.3118x; 1.1189x over previous
import functools

import jax
import jax.numpy as jnp
from jax import lax
from jax.experimental import pallas as pl
from jax.experimental.pallas import tpu as pltpu
from jax.experimental.pallas import tpu_sc as plsc

NC = 2
NS = 16
L = 16


def _sc_body(B, C, HW, ML, no, NOBJ,
             ft_hbm, lab_hbm, nobj_hbm,
             src_obj, mask_obj, src_bg, mask_bg, src_other, mask_other,
             labv, idxbuf0, idxbuf1, maskv0, maskv1, nobjv, fbuf, sem_g, sem_s):
    cid = lax.axis_index("c")
    sid = lax.axis_index("s")
    wid = sid * NC + cid
    pltpu.sync_copy(nobj_hbm, nobjv)
    lanes = lax.broadcasted_iota(jnp.int32, (L,), 0)
    nchunk = HW // L
    ccol = C // L

    zf32 = jnp.zeros((L,), jnp.float32)
    for cc in range(ccol):
        fbuf[HW, pl.ds(cc * L, L)] = zf32

    b = wid // 4
    cpf = pltpu.async_copy(ft_hbm.at[pl.ds(b * HW, HW)],
                           fbuf.at[pl.ds(0, HW)], sem_g)

    counts = []
    zfill = jnp.full((L,), HW, jnp.int32)
    for r in range(2):
        idxbuf = idxbuf0 if r == 0 else idxbuf1
        maskv = maskv0 if r == 0 else maskv1
        t = wid * 2 + r
        chm1 = t % 8
        ch = chm1 + 1
        pltpu.sync_copy(lab_hbm.at[pl.ds((b * NOBJ + ch) * HW, HW)], labv)
        nobj_b = jnp.sum(jnp.where(lanes == b, nobjv[...], 0))
        skip = jnp.logical_and(chm1 < 7, ch > nobj_b)
        keep = jnp.broadcast_to(jnp.logical_not(skip), (L,))

        for k in range(nchunk):
            idxbuf[pl.ds(k * L, L)] = zfill
        off = jnp.int32(0)
        for k in range(nchunk):
            lab16 = labv[pl.ds(k * L, L)]
            m = jnp.logical_and(lab16 == 1, keep)
            mi = m.astype(jnp.int32)
            pos = plsc.cumsum(mi) - mi + off
            pix = lanes + (k * L)
            plsc.store_scatter(idxbuf, [pos], pix, mask=m)
            off = off + jnp.sum(mi)
        counts.append(off)

        for k in range(nchunk):
            i16 = lanes + (k * L)
            maskv[pl.ds(k * L, L)] = jnp.where(
                i16 >= off, jnp.float32(1.0), jnp.float32(0.0))

    for r in range(2):
        idxbuf = idxbuf0 if r == 0 else idxbuf1
        maskv = maskv0 if r == 0 else maskv1
        t = wid * 2 + r
        chm1 = t % 8
        is_bg = chm1 == 7
        count = counts[r]
        if r == 1:
            cpf = pltpu.async_copy(ft_hbm.at[pl.ds(b * HW, HW)],
                                   fbuf.at[pl.ds(0, HW)], sem_g)
        cpf.wait()

        def _pack_chunk(k, _):
            i0 = k * L
            idx16 = idxbuf[pl.ds(i0, L)]
            for kk in range(L):
                si = idx16[kk]
                for cc in range(ccol):
                    fbuf[i0 + kk, pl.ds(cc * L, L)] = fbuf[si, pl.ds(cc * L, L)]
            return 0

        def _zero_chunk(k, _):
            i0 = k * L
            for kk in range(L):
                for cc in range(ccol):
                    fbuf[i0 + kk, pl.ds(cc * L, L)] = zf32
            return 0

        kc = (count + (L - 1)) // L
        lax.fori_loop(0, kc, _pack_chunk, 0)
        lax.fori_loop(kc, nchunk, _zero_chunk, 0)

        o = chm1

        @pl.when(jnp.logical_not(is_bg))
        def _():
            pend = []
            base0 = (b * no + o) * ML
            pend.append(pltpu.async_copy(
                fbuf.at[pl.ds(0, ML)], src_obj.at[pl.ds(base0, ML)], sem_s))
            pend.append(pltpu.async_copy(
                maskv, mask_obj.at[pl.ds(base0, ML)], sem_s))
            for d in range(1, no):
                o2 = lax.rem(o + d, no)
                j = o - (o2 < o).astype(jnp.int32)
                base = (b * no + o2) * (ML * no) + j * ML
                pend.append(pltpu.async_copy(
                    fbuf.at[pl.ds(0, ML)], src_other.at[pl.ds(base, ML)], sem_s))
                pend.append(pltpu.async_copy(
                    maskv, mask_other.at[pl.ds(base, ML)], sem_s))
            for p in pend:
                p.wait()

        @pl.when(is_bg)
        def _():
            pend = []
            pend.append(pltpu.async_copy(
                fbuf.at[pl.ds(0, ML)], src_bg.at[pl.ds(b * ML, ML)], sem_s))
            pend.append(pltpu.async_copy(
                maskv, mask_bg.at[pl.ds(b * ML, ML)], sem_s))
            for o2 in range(no):
                base = (b * no + o2) * (ML * no) + (no - 1) * ML
                pend.append(pltpu.async_copy(
                    fbuf.at[pl.ds(0, ML)], src_other.at[pl.ds(base, ML)], sem_s))
                pend.append(pltpu.async_copy(
                    maskv, mask_other.at[pl.ds(base, ML)], sem_s))
            for p in pend:
                p.wait()


def kernel(feats, label, num_objects):
    B, C, H, W = feats.shape
    HW = H * W
    ML = 256
    NOBJ = label.shape[1]
    no = num_objects.shape[0] - 1

    ft = feats.reshape(B, C, HW).transpose(0, 2, 1).reshape(B * HW, C)
    lab_flat = label.reshape(B * NOBJ * HW).astype(jnp.int32)
    nobj16 = jnp.pad(num_objects.astype(jnp.int32), (0, 16 - B))

    mesh = plsc.VectorSubcoreMesh(core_axis_name="c", subcore_axis_name="s",
                                  num_cores=NC, num_subcores=NS)
    out_type = (
        jax.ShapeDtypeStruct((B * no * ML, C), jnp.float32),
        jax.ShapeDtypeStruct((B * no * ML,), jnp.float32),
        jax.ShapeDtypeStruct((B * ML, C), jnp.float32),
        jax.ShapeDtypeStruct((B * ML,), jnp.float32),
        jax.ShapeDtypeStruct((B * no * ML * no, C), jnp.float32),
        jax.ShapeDtypeStruct((B * no * ML * no,), jnp.float32),
    )
    scratch_types = [
        pltpu.VMEM((HW,), jnp.int32),
        pltpu.VMEM((HW + L,), jnp.int32),
        pltpu.VMEM((HW + L,), jnp.int32),
        pltpu.VMEM((HW,), jnp.float32),
        pltpu.VMEM((HW,), jnp.float32),
        pltpu.VMEM((16,), jnp.int32),
        pltpu.VMEM((HW + 1, C), jnp.float32),
        pltpu.SemaphoreType.DMA,
        pltpu.SemaphoreType.DMA,
    ]
    body = functools.partial(_sc_body, B, C, HW, ML, no, NOBJ)
    outs = pl.kernel(
        body, out_type=out_type, mesh=mesh,
        scratch_types=scratch_types,
        compiler_params=pltpu.CompilerParams(needs_layout_passes=False),
        name="scribble_pool_sc")(ft, lab_flat, nobj16)
    o1, o2, o3, o4, o5, o6 = outs
    return (o1.reshape(B * no, ML, C),
            o2.reshape(B * no, ML),
            o3.reshape(B, ML, C),
            o4.reshape(B, ML),
            o5.reshape(B * no, ML * no, C),
            o6.reshape(B * no, ML * no))


# speedup vs baseline: 2.3122x; 1.0002x over previous
import functools

import jax
import jax.numpy as jnp
from jax import lax
from jax.experimental import pallas as pl
from jax.experimental.pallas import tpu as pltpu
from jax.experimental.pallas import tpu_sc as plsc

NC = 2
NS = 16
L = 16


def _sc_body(B, C, HW, ML, no, NOBJ,
             ft_hbm, lab_hbm, nobj_hbm,
             src_obj, mask_obj, src_bg, mask_bg, src_other, mask_other,
             labv, idxbuf0, idxbuf1, maskv0, maskv1, nobjv, fbuf,
             sem_g, sem_l, sem_s):
    cid = lax.axis_index("c")
    sid = lax.axis_index("s")
    wid = sid * NC + cid
    pltpu.sync_copy(nobj_hbm, nobjv)
    lanes = lax.broadcasted_iota(jnp.int32, (L,), 0)
    nchunk = HW // L
    ccol = C // L

    zf32 = jnp.zeros((L,), jnp.float32)
    for cc in range(ccol):
        fbuf[HW, pl.ds(cc * L, L)] = zf32

    b = wid // 4
    cpf = pltpu.async_copy(ft_hbm.at[pl.ds(b * HW, HW)],
                           fbuf.at[pl.ds(0, HW)], sem_g)
    chm1_0 = (wid * 2) % 8
    cpl = pltpu.async_copy(
        lab_hbm.at[pl.ds((b * NOBJ + chm1_0 + 1) * HW, 2 * HW)], labv, sem_l)

    counts = []
    zfill = jnp.full((L,), HW, jnp.int32)
    for r in range(2):
        idxbuf = idxbuf0 if r == 0 else idxbuf1
        maskv = maskv0 if r == 0 else maskv1
        t = wid * 2 + r
        chm1 = t % 8
        ch = chm1 + 1
        nobj_b = jnp.sum(jnp.where(lanes == b, nobjv[...], 0))
        skip = jnp.logical_and(chm1 < 7, ch > nobj_b)
        keep = jnp.broadcast_to(jnp.logical_not(skip), (L,))

        for k in range(nchunk):
            idxbuf[pl.ds(k * L, L)] = zfill
        if r == 0:
            cpl.wait()
        off = jnp.int32(0)
        for k in range(nchunk):
            lab16 = labv[pl.ds(r * HW + k * L, L)]
            m = jnp.logical_and(lab16 == 1, keep)
            mi = m.astype(jnp.int32)
            pos = plsc.cumsum(mi) - mi + off
            pix = lanes + (k * L)
            plsc.store_scatter(idxbuf, [pos], pix, mask=m)
            off = off + jnp.sum(mi)
        counts.append(off)

        for k in range(nchunk):
            i16 = lanes + (k * L)
            maskv[pl.ds(k * L, L)] = jnp.where(
                i16 >= off, jnp.float32(1.0), jnp.float32(0.0))

    for r in range(2):
        idxbuf = idxbuf0 if r == 0 else idxbuf1
        maskv = maskv0 if r == 0 else maskv1
        t = wid * 2 + r
        chm1 = t % 8
        is_bg = chm1 == 7
        count = counts[r]
        if r == 1:
            cpf = pltpu.async_copy(ft_hbm.at[pl.ds(b * HW, HW)],
                                   fbuf.at[pl.ds(0, HW)], sem_g)
        cpf.wait()

        def _pack_chunk(k, _):
            i0 = k * L
            idx16 = idxbuf[pl.ds(i0, L)]
            for kk in range(L):
                si = idx16[kk]
                for cc in range(ccol):
                    fbuf[i0 + kk, pl.ds(cc * L, L)] = fbuf[si, pl.ds(cc * L, L)]
            return 0

        def _zero_chunk(k, _):
            i0 = k * L
            for kk in range(L):
                for cc in range(ccol):
                    fbuf[i0 + kk, pl.ds(cc * L, L)] = zf32
            return 0

        kc = (count + (L - 1)) // L
        lax.fori_loop(0, kc, _pack_chunk, 0)
        lax.fori_loop(kc, nchunk, _zero_chunk, 0)

        o = chm1

        @pl.when(jnp.logical_not(is_bg))
        def _():
            pend = []
            base0 = (b * no + o) * ML
            pend.append(pltpu.async_copy(
                fbuf.at[pl.ds(0, ML)], src_obj.at[pl.ds(base0, ML)], sem_s))
            pend.append(pltpu.async_copy(
                maskv, mask_obj.at[pl.ds(base0, ML)], sem_s))
            for d in range(1, no):
                o2 = lax.rem(o + d, no)
                j = o - (o2 < o).astype(jnp.int32)
                base = (b * no + o2) * (ML * no) + j * ML
                pend.append(pltpu.async_copy(
                    fbuf.at[pl.ds(0, ML)], src_other.at[pl.ds(base, ML)], sem_s))
                pend.append(pltpu.async_copy(
                    maskv, mask_other.at[pl.ds(base, ML)], sem_s))
            for p in pend:
                p.wait()

        @pl.when(is_bg)
        def _():
            pend = []
            pend.append(pltpu.async_copy(
                fbuf.at[pl.ds(0, ML)], src_bg.at[pl.ds(b * ML, ML)], sem_s))
            pend.append(pltpu.async_copy(
                maskv, mask_bg.at[pl.ds(b * ML, ML)], sem_s))
            for o2 in range(no):
                base = (b * no + o2) * (ML * no) + (no - 1) * ML
                pend.append(pltpu.async_copy(
                    fbuf.at[pl.ds(0, ML)], src_other.at[pl.ds(base, ML)], sem_s))
                pend.append(pltpu.async_copy(
                    maskv, mask_other.at[pl.ds(base, ML)], sem_s))
            for p in pend:
                p.wait()


def kernel(feats, label, num_objects):
    B, C, H, W = feats.shape
    HW = H * W
    ML = 256
    NOBJ = label.shape[1]
    no = num_objects.shape[0] - 1

    ft = feats.reshape(B, C, HW).transpose(0, 2, 1).reshape(B * HW, C)
    lab_flat = label.reshape(B * NOBJ * HW).astype(jnp.int32)
    nobj16 = jnp.pad(num_objects.astype(jnp.int32), (0, 16 - B))

    mesh = plsc.VectorSubcoreMesh(core_axis_name="c", subcore_axis_name="s",
                                  num_cores=NC, num_subcores=NS)
    out_type = (
        jax.ShapeDtypeStruct((B * no * ML, C), jnp.float32),
        jax.ShapeDtypeStruct((B * no * ML,), jnp.float32),
        jax.ShapeDtypeStruct((B * ML, C), jnp.float32),
        jax.ShapeDtypeStruct((B * ML,), jnp.float32),
        jax.ShapeDtypeStruct((B * no * ML * no, C), jnp.float32),
        jax.ShapeDtypeStruct((B * no * ML * no,), jnp.float32),
    )
    scratch_types = [
        pltpu.VMEM((2 * HW,), jnp.int32),
        pltpu.VMEM((HW + L,), jnp.int32),
        pltpu.VMEM((HW + L,), jnp.int32),
        pltpu.VMEM((HW,), jnp.float32),
        pltpu.VMEM((HW,), jnp.float32),
        pltpu.VMEM((16,), jnp.int32),
        pltpu.VMEM((HW + 1, C), jnp.float32),
        pltpu.SemaphoreType.DMA,
        pltpu.SemaphoreType.DMA,
        pltpu.SemaphoreType.DMA,
    ]
    body = functools.partial(_sc_body, B, C, HW, ML, no, NOBJ)
    outs = pl.kernel(
        body, out_type=out_type, mesh=mesh,
        scratch_types=scratch_types,
        compiler_params=pltpu.CompilerParams(needs_layout_passes=False),
        name="scribble_pool_sc")(ft, lab_flat, nobj16)
    o1, o2, o3, o4, o5, o6 = outs
    return (o1.reshape(B * no, ML, C),
            o2.reshape(B * no, ML),
            o3.reshape(B, ML, C),
            o4.reshape(B, ML),
            o5.reshape(B * no, ML * no, C),
            o6.reshape(B * no, ML * no))
